# Initial kernel scaffold; baseline (speedup 1.0000x reference)
import jax
import jax.numpy as jnp
from jax.experimental import pallas as pl


def kernel(predicted_locs, predicted_scores, boxes, labels, priors_cxcy):
    raise NotImplementedError("write your pallas kernel here")


# speedup vs baseline: 2.5206x; 2.5206x over previous
import jax
import jax.numpy as jnp
from jax import lax
from jax.experimental import pallas as pl
from jax.experimental.pallas import tpu as pltpu

B = 8
N = 22536
NOBJ = 16
C = 80
THRESHOLD = 0.5
GAMMA = 2.0
ALPHA_F = 0.25
TP = 2504
NT = N // TP

_f32 = jnp.float32
_i32 = jnp.int32


def _match_body(boxes_ref, labels_ref, priors_ref, plocs_ref,
                tcls_ref, loc_ref, npos_ref):
    b = pl.program_id(0)

    pr = priors_ref[...]
    pcx, pcy = pr[0:1], pr[1:2]
    pw, ph = pr[2:3], pr[3:4]
    px0 = pcx - pw * 0.5
    py0 = pcy - ph * 0.5
    px1 = pcx + pw * 0.5
    py1 = pcy + ph * 0.5

    bx = boxes_ref[0]
    bx0, by0 = bx[:, 0:1], bx[:, 1:2]
    bx1, by1 = bx[:, 2:3], bx[:, 3:4]

    iw = jnp.maximum(jnp.minimum(bx1, px1) - jnp.maximum(bx0, px0), 0.0)
    ih = jnp.maximum(jnp.minimum(by1, py1) - jnp.maximum(by0, py0), 0.0)
    inter = iw * ih
    barea = (bx1 - bx0) * (by1 - by0)
    parea = (px1 - px0) * (py1 - py0)
    ov = inter / (barea + parea - inter)

    jj = lax.broadcasted_iota(_i32, (NOBJ, 1), 0)
    nn = lax.broadcasted_iota(_i32, (1, N), 1)

    ofp = jnp.max(ov, axis=0, keepdims=True)
    is_max = ov == ofp
    obj = jnp.min(jnp.where(is_max, jj, NOBJ), axis=0, keepdims=True)

    oeo = jnp.max(ov, axis=1, keepdims=True)
    pfe = jnp.min(jnp.where(ov == oeo, nn, N), axis=1, keepdims=True)
    maskj = oeo > 0.0

    cond = (nn == pfe) & maskj
    bestj = jnp.max(jnp.where(cond, jj, -1), axis=0, keepdims=True)
    forced = bestj >= 0
    ofp = jnp.where(forced, 1.0, ofp)
    obj = jnp.where(forced, bestj, obj)

    sel = jj == obj
    lab = jnp.sum(jnp.where(sel, labels_ref[0], 0), axis=0, keepdims=True)
    gx0 = jnp.sum(jnp.where(sel, bx0, 0.0), axis=0, keepdims=True)
    gy0 = jnp.sum(jnp.where(sel, by0, 0.0), axis=0, keepdims=True)
    gx1 = jnp.sum(jnp.where(sel, bx1, 0.0), axis=0, keepdims=True)
    gy1 = jnp.sum(jnp.where(sel, by1, 0.0), axis=0, keepdims=True)

    lab = jnp.where(ofp < THRESHOLD, -1, lab)
    lab = jnp.where(ofp < THRESHOLD - 0.1, 0, lab)

    ex = (0.5 * (gx0 + gx1) - pcx) / (pw / 10.0)
    ey = (0.5 * (gy0 + gy1) - pcy) / (ph / 10.0)
    ew = jnp.log(jnp.maximum((gx1 - gx0) / pw, 1e-12)) * 5.0
    eh = jnp.log(jnp.maximum((gy1 - gy0) / ph, 1e-12)) * 5.0
    enc = jnp.concatenate([ex, ey, ew, eh], axis=0)

    posf = (lab > 0).astype(_f32)
    d = plocs_ref[0] - enc
    ad = jnp.abs(d)
    sl1 = jnp.where(ad < 1.0, 0.5 * d * d, ad - 0.5)
    loc_part = jnp.sum(sl1 * posf)
    npos_part = jnp.sum(posf)

    tcls_ref[0] = lab

    @pl.when(b == 0)
    def _():
        loc_ref[0, 0] = 0.0
        npos_ref[0, 0] = 0.0

    loc_ref[0, 0] += loc_part
    npos_ref[0, 0] += npos_part


def _conf_body(scores_ref, tcls_ref, out_ref):
    b = pl.program_id(0)
    t = pl.program_id(1)

    s = scores_ref[0]
    tc = tcls_ref[0]
    cr = lax.broadcasted_iota(_i32, (1, C), 1) + 1
    pos = tc == cr
    neg = (~pos) & (tc >= 0)

    p = 1.0 / (1.0 + jnp.exp(-s))
    omp = 1.0 - p
    term1 = omp * omp * jnp.log(jnp.maximum(p, 1e-12))
    term2 = p * p * jnp.log(jnp.maximum(omp, 1e-12))
    loss = (jnp.where(pos, -ALPHA_F * term1, 0.0)
            + jnp.where(neg, -(1.0 - ALPHA_F) * term2, 0.0))
    part = jnp.sum(loss)

    @pl.when(jnp.logical_and(b == 0, t == 0))
    def _():
        out_ref[0, 0] = 0.0

    out_ref[0, 0] += part


def kernel(predicted_locs, predicted_scores, boxes, labels, priors_cxcy):
    labels3 = labels.astype(_i32).reshape(B, NOBJ, 1)
    priors_t = priors_cxcy.T
    plocs_t = jnp.transpose(predicted_locs, (0, 2, 1))

    tcls, loc_sum, npos = pl.pallas_call(
        _match_body,
        grid=(B,),
        in_specs=[
            pl.BlockSpec((1, NOBJ, 4), lambda b: (b, 0, 0)),
            pl.BlockSpec((1, NOBJ, 1), lambda b: (b, 0, 0)),
            pl.BlockSpec((4, N), lambda b: (0, 0)),
            pl.BlockSpec((1, 4, N), lambda b: (b, 0, 0)),
        ],
        out_specs=[
            pl.BlockSpec((1, 1, N), lambda b: (b, 0, 0)),
            pl.BlockSpec((1, 1), lambda b: (0, 0), memory_space=pltpu.SMEM),
            pl.BlockSpec((1, 1), lambda b: (0, 0), memory_space=pltpu.SMEM),
        ],
        out_shape=[
            jax.ShapeDtypeStruct((B, 1, N), _i32),
            jax.ShapeDtypeStruct((1, 1), _f32),
            jax.ShapeDtypeStruct((1, 1), _f32),
        ],
    )(boxes, labels3, priors_t, plocs_t)

    tcls3 = jnp.transpose(tcls, (0, 2, 1))

    conf = pl.pallas_call(
        _conf_body,
        grid=(B, NT),
        in_specs=[
            pl.BlockSpec((1, TP, C), lambda b, t: (b, t, 0)),
            pl.BlockSpec((1, TP, 1), lambda b, t: (b, t, 0)),
        ],
        out_specs=pl.BlockSpec((1, 1), lambda b, t: (0, 0),
                               memory_space=pltpu.SMEM),
        out_shape=jax.ShapeDtypeStruct((1, 1), _f32),
    )(predicted_scores, tcls3)

    np_ = npos[0, 0]
    return conf[0, 0] / np_ + loc_sum[0, 0] / (np_ * 4.0)


# speedup vs baseline: 2.6707x; 1.0596x over previous
import jax
import jax.numpy as jnp
from jax import lax
from jax.experimental import pallas as pl
from jax.experimental.pallas import tpu as pltpu

B = 8
N = 22536
NOBJ = 16
C = 80
THRESHOLD = 0.5
GAMMA = 2.0
ALPHA_F = 0.25
TP = 7512
NT = N // TP

_f32 = jnp.float32
_i32 = jnp.int32


def _match_body(boxes_ref, labels_ref, priors_ref, plocs_ref,
                tcls_ref, loc_ref, npos_ref):
    b = pl.program_id(0)

    pr = priors_ref[...]
    pcx, pcy = pr[0:1], pr[1:2]
    pw, ph = pr[2:3], pr[3:4]
    px0 = pcx - pw * 0.5
    py0 = pcy - ph * 0.5
    px1 = pcx + pw * 0.5
    py1 = pcy + ph * 0.5

    bx = boxes_ref[0]
    bx0, by0 = bx[:, 0:1], bx[:, 1:2]
    bx1, by1 = bx[:, 2:3], bx[:, 3:4]

    iw = jnp.maximum(jnp.minimum(bx1, px1) - jnp.maximum(bx0, px0), 0.0)
    ih = jnp.maximum(jnp.minimum(by1, py1) - jnp.maximum(by0, py0), 0.0)
    inter = iw * ih
    barea = (bx1 - bx0) * (by1 - by0)
    parea = (px1 - px0) * (py1 - py0)
    ov = inter / (barea + parea - inter)

    jj = lax.broadcasted_iota(_i32, (NOBJ, 1), 0)
    nn = lax.broadcasted_iota(_i32, (1, N), 1)

    ofp = jnp.max(ov, axis=0, keepdims=True)
    is_max = ov == ofp
    obj = jnp.min(jnp.where(is_max, jj, NOBJ), axis=0, keepdims=True)

    oeo = jnp.max(ov, axis=1, keepdims=True)
    pfe = jnp.min(jnp.where(ov == oeo, nn, N), axis=1, keepdims=True)
    maskj = oeo > 0.0

    cond = (nn == pfe) & maskj
    bestj = jnp.max(jnp.where(cond, jj, -1), axis=0, keepdims=True)
    forced = bestj >= 0
    ofp = jnp.where(forced, 1.0, ofp)
    obj = jnp.where(forced, bestj, obj)

    sel = jj == obj
    lab = jnp.sum(jnp.where(sel, labels_ref[0], 0), axis=0, keepdims=True)
    gx0 = jnp.sum(jnp.where(sel, bx0, 0.0), axis=0, keepdims=True)
    gy0 = jnp.sum(jnp.where(sel, by0, 0.0), axis=0, keepdims=True)
    gx1 = jnp.sum(jnp.where(sel, bx1, 0.0), axis=0, keepdims=True)
    gy1 = jnp.sum(jnp.where(sel, by1, 0.0), axis=0, keepdims=True)

    lab = jnp.where(ofp < THRESHOLD, -1, lab)
    lab = jnp.where(ofp < THRESHOLD - 0.1, 0, lab)

    ex = (0.5 * (gx0 + gx1) - pcx) / (pw / 10.0)
    ey = (0.5 * (gy0 + gy1) - pcy) / (ph / 10.0)
    ew = jnp.log(jnp.maximum((gx1 - gx0) / pw, 1e-12)) * 5.0
    eh = jnp.log(jnp.maximum((gy1 - gy0) / ph, 1e-12)) * 5.0
    enc = jnp.concatenate([ex, ey, ew, eh], axis=0)

    posf = (lab > 0).astype(_f32)
    d = plocs_ref[0] - enc
    ad = jnp.abs(d)
    sl1 = jnp.where(ad < 1.0, 0.5 * d * d, ad - 0.5)
    loc_part = jnp.sum(sl1 * posf)
    npos_part = jnp.sum(posf)

    tcls_ref[0] = lab

    @pl.when(b == 0)
    def _():
        loc_ref[0, 0] = 0.0
        npos_ref[0, 0] = 0.0

    loc_ref[0, 0] += loc_part
    npos_ref[0, 0] += npos_part


def _focal_tile(s, tc):
    cr = lax.broadcasted_iota(_i32, (1, C), 1) + 1
    pos = tc == cr
    neg = (~pos) & (tc >= 0)

    p = 1.0 / (1.0 + jnp.exp(-s))
    omp = 1.0 - p
    term1 = omp * omp * jnp.log(jnp.maximum(p, 1e-12))
    term2 = p * p * jnp.log(jnp.maximum(omp, 1e-12))
    loss = (jnp.where(pos, -ALPHA_F * term1, 0.0)
            + jnp.where(neg, -(1.0 - ALPHA_F) * term2, 0.0))
    return jnp.sum(loss)


def _conf_body(s1_ref, s2_ref, t1_ref, t2_ref, out_ref):
    b = pl.program_id(0)
    t = pl.program_id(1)

    part = (_focal_tile(s1_ref[0], t1_ref[0])
            + _focal_tile(s2_ref[0], t2_ref[0]))

    @pl.when(jnp.logical_and(b == 0, t == 0))
    def _():
        out_ref[0, 0] = 0.0

    out_ref[0, 0] += part


def kernel(predicted_locs, predicted_scores, boxes, labels, priors_cxcy):
    labels3 = labels.astype(_i32).reshape(B, NOBJ, 1)
    priors_t = priors_cxcy.T
    plocs_t = jnp.transpose(predicted_locs, (0, 2, 1))

    tcls, loc_sum, npos = pl.pallas_call(
        _match_body,
        grid=(B,),
        in_specs=[
            pl.BlockSpec((1, NOBJ, 4), lambda b: (b, 0, 0)),
            pl.BlockSpec((1, NOBJ, 1), lambda b: (b, 0, 0)),
            pl.BlockSpec((4, N), lambda b: (0, 0)),
            pl.BlockSpec((1, 4, N), lambda b: (b, 0, 0)),
        ],
        out_specs=[
            pl.BlockSpec((1, 1, N), lambda b: (b, 0, 0)),
            pl.BlockSpec((1, 1), lambda b: (0, 0), memory_space=pltpu.SMEM),
            pl.BlockSpec((1, 1), lambda b: (0, 0), memory_space=pltpu.SMEM),
        ],
        out_shape=[
            jax.ShapeDtypeStruct((B, 1, N), _i32),
            jax.ShapeDtypeStruct((1, 1), _f32),
            jax.ShapeDtypeStruct((1, 1), _f32),
        ],
    )(boxes, labels3, priors_t, plocs_t)

    tcls3 = jnp.transpose(tcls, (0, 2, 1))

    conf = pl.pallas_call(
        _conf_body,
        grid=(B // 2, NT),
        in_specs=[
            pl.BlockSpec((1, TP, C), lambda b, t: (b, t, 0)),
            pl.BlockSpec((1, TP, C), lambda b, t: (b + B // 2, t, 0)),
            pl.BlockSpec((1, TP, 1), lambda b, t: (b, t, 0)),
            pl.BlockSpec((1, TP, 1), lambda b, t: (b + B // 2, t, 0)),
        ],
        out_specs=pl.BlockSpec((1, 1), lambda b, t: (0, 0),
                               memory_space=pltpu.SMEM),
        out_shape=jax.ShapeDtypeStruct((1, 1), _f32),
    )(predicted_scores, predicted_scores, tcls3, tcls3)

    np_ = npos[0, 0]
    return conf[0, 0] / np_ + loc_sum[0, 0] / (np_ * 4.0)


# speedup vs baseline: 3.1901x; 1.1945x over previous
import jax
import jax.numpy as jnp
from jax import lax
from jax.experimental import pallas as pl
from jax.experimental.pallas import tpu as pltpu

B = 8
N = 22536
NOBJ = 16
C = 80
THRESHOLD = 0.5
GAMMA = 2.0
ALPHA_F = 0.25
TP = 7512
NT = N // TP

_f32 = jnp.float32
_i32 = jnp.int32


def _match_body(boxes_ref, labels_ref, priors_ref, plocs_ref,
                tcls_ref, loc_ref, npos_ref):
    b = pl.program_id(0)

    pr = priors_ref[...]
    pcx, pcy = pr[0:1], pr[1:2]
    pw, ph = pr[2:3], pr[3:4]
    px0 = pcx - pw * 0.5
    py0 = pcy - ph * 0.5
    px1 = pcx + pw * 0.5
    py1 = pcy + ph * 0.5

    bx = boxes_ref[0]
    bx0, by0 = bx[:, 0:1], bx[:, 1:2]
    bx1, by1 = bx[:, 2:3], bx[:, 3:4]

    iw = jnp.maximum(jnp.minimum(bx1, px1) - jnp.maximum(bx0, px0), 0.0)
    ih = jnp.maximum(jnp.minimum(by1, py1) - jnp.maximum(by0, py0), 0.0)
    inter = iw * ih
    barea = (bx1 - bx0) * (by1 - by0)
    parea = (px1 - px0) * (py1 - py0)
    ov = inter / (barea + parea - inter)

    jj = lax.broadcasted_iota(_i32, (NOBJ, 1), 0)
    nn = lax.broadcasted_iota(_i32, (1, N), 1)

    ofp = jnp.max(ov, axis=0, keepdims=True)
    is_max = ov == ofp
    obj = jnp.min(jnp.where(is_max, jj, NOBJ), axis=0, keepdims=True)

    oeo = jnp.max(ov, axis=1, keepdims=True)
    pfe = jnp.min(jnp.where(ov == oeo, nn, N), axis=1, keepdims=True)
    maskj = oeo > 0.0

    cond = (nn == pfe) & maskj
    bestj = jnp.max(jnp.where(cond, jj, -1), axis=0, keepdims=True)
    forced = bestj >= 0
    ofp = jnp.where(forced, 1.0, ofp)
    obj = jnp.where(forced, bestj, obj)

    sel = jj == obj
    lab = jnp.sum(jnp.where(sel, labels_ref[0], 0), axis=0, keepdims=True)
    gx0 = jnp.sum(jnp.where(sel, bx0, 0.0), axis=0, keepdims=True)
    gy0 = jnp.sum(jnp.where(sel, by0, 0.0), axis=0, keepdims=True)
    gx1 = jnp.sum(jnp.where(sel, bx1, 0.0), axis=0, keepdims=True)
    gy1 = jnp.sum(jnp.where(sel, by1, 0.0), axis=0, keepdims=True)

    lab = jnp.where(ofp < THRESHOLD, -1, lab)
    lab = jnp.where(ofp < THRESHOLD - 0.1, 0, lab)

    ex = (0.5 * (gx0 + gx1) - pcx) / (pw / 10.0)
    ey = (0.5 * (gy0 + gy1) - pcy) / (ph / 10.0)
    ew = jnp.log(jnp.maximum((gx1 - gx0) / pw, 1e-12)) * 5.0
    eh = jnp.log(jnp.maximum((gy1 - gy0) / ph, 1e-12)) * 5.0
    enc = jnp.concatenate([ex, ey, ew, eh], axis=0)

    posf = (lab > 0).astype(_f32)
    d = plocs_ref[0] - enc
    ad = jnp.abs(d)
    sl1 = jnp.where(ad < 1.0, 0.5 * d * d, ad - 0.5)
    loc_part = jnp.sum(sl1 * posf)
    npos_part = jnp.sum(posf)

    tcls_ref[0] = lab

    @pl.when(b == 0)
    def _():
        loc_ref[0, 0] = 0.0
        npos_ref[0, 0] = 0.0

    loc_ref[0, 0] += loc_part
    npos_ref[0, 0] += npos_part


def _focal_tile(s, tc):
    cr = lax.broadcasted_iota(_i32, (1, C), 1) + 1
    pos = tc == cr
    neg = (~pos) & (tc >= 0)

    p = 1.0 / (1.0 + jnp.exp(-s))
    omp = 1.0 - p
    term1 = omp * omp * jnp.log(jnp.maximum(p, 1e-12))
    term2 = p * p * jnp.log(jnp.maximum(omp, 1e-12))
    loss = (jnp.where(pos, -ALPHA_F * term1, 0.0)
            + jnp.where(neg, -(1.0 - ALPHA_F) * term2, 0.0))
    return jnp.sum(loss)


def _conf_body(s1_ref, s2_ref, t1_ref, t2_ref, out_ref, tsc1, tsc2):
    b = pl.program_id(0)
    t = pl.program_id(1)

    @pl.when(t == 0)
    def _():
        tsc1[...] = jnp.transpose(t1_ref[0])
        tsc2[...] = jnp.transpose(t2_ref[0])

    off = pl.multiple_of(t * TP, 8)
    t1 = tsc1[pl.ds(off, TP), :]
    t2 = tsc2[pl.ds(off, TP), :]
    part = (_focal_tile(s1_ref[0], t1)
            + _focal_tile(s2_ref[0], t2))

    @pl.when(jnp.logical_and(b == 0, t == 0))
    def _():
        out_ref[0, 0] = 0.0

    out_ref[0, 0] += part


def kernel(predicted_locs, predicted_scores, boxes, labels, priors_cxcy):
    labels3 = labels.astype(_i32).reshape(B, NOBJ, 1)
    priors_t = priors_cxcy.T
    plocs_t = jnp.transpose(predicted_locs, (0, 2, 1))

    tcls, loc_sum, npos = pl.pallas_call(
        _match_body,
        grid=(B,),
        in_specs=[
            pl.BlockSpec((1, NOBJ, 4), lambda b: (b, 0, 0)),
            pl.BlockSpec((1, NOBJ, 1), lambda b: (b, 0, 0)),
            pl.BlockSpec((4, N), lambda b: (0, 0)),
            pl.BlockSpec((1, 4, N), lambda b: (b, 0, 0)),
        ],
        out_specs=[
            pl.BlockSpec((1, 1, N), lambda b: (b, 0, 0)),
            pl.BlockSpec((1, 1), lambda b: (0, 0), memory_space=pltpu.SMEM),
            pl.BlockSpec((1, 1), lambda b: (0, 0), memory_space=pltpu.SMEM),
        ],
        out_shape=[
            jax.ShapeDtypeStruct((B, 1, N), _i32),
            jax.ShapeDtypeStruct((1, 1), _f32),
            jax.ShapeDtypeStruct((1, 1), _f32),
        ],
    )(boxes, labels3, priors_t, plocs_t)

    conf = pl.pallas_call(
        _conf_body,
        grid=(B // 2, NT),
        in_specs=[
            pl.BlockSpec((1, TP, C), lambda b, t: (b, t, 0)),
            pl.BlockSpec((1, TP, C), lambda b, t: (b + B // 2, t, 0)),
            pl.BlockSpec((1, 1, N), lambda b, t: (b, 0, 0)),
            pl.BlockSpec((1, 1, N), lambda b, t: (b + B // 2, 0, 0)),
        ],
        out_specs=pl.BlockSpec((1, 1), lambda b, t: (0, 0),
                               memory_space=pltpu.SMEM),
        out_shape=jax.ShapeDtypeStruct((1, 1), _f32),
        scratch_shapes=[pltpu.VMEM((N, 1), _i32), pltpu.VMEM((N, 1), _i32)],
    )(predicted_scores, predicted_scores, tcls, tcls)

    np_ = npos[0, 0]
    return conf[0, 0] / np_ + loc_sum[0, 0] / (np_ * 4.0)


# speedup vs baseline: 3.2316x; 1.0130x over previous
import jax
import jax.numpy as jnp
from jax import lax
from jax.experimental import pallas as pl
from jax.experimental.pallas import tpu as pltpu

B = 8
N = 22536
NOBJ = 16
C = 80
THRESHOLD = 0.5
ALPHA_F = 0.25
TP = 7512
NT = N // TP

_f32 = jnp.float32
_i32 = jnp.int32
_bf16 = jnp.bfloat16


def _match_body(boxes_ref, labels_ref, priors_ref, plocs_ref,
                tcls_ref, loc_ref, npos_ref):
    b = pl.program_id(0)

    pr = priors_ref[...]
    pcx, pcy = pr[0:1], pr[1:2]
    pw, ph = pr[2:3], pr[3:4]
    px0 = pcx - pw * 0.5
    py0 = pcy - ph * 0.5
    px1 = pcx + pw * 0.5
    py1 = pcy + ph * 0.5

    bx = boxes_ref[0]
    bx0, by0 = bx[:, 0:1], bx[:, 1:2]
    bx1, by1 = bx[:, 2:3], bx[:, 3:4]

    iw = jnp.maximum(jnp.minimum(bx1, px1) - jnp.maximum(bx0, px0), 0.0)
    ih = jnp.maximum(jnp.minimum(by1, py1) - jnp.maximum(by0, py0), 0.0)
    inter = iw * ih
    barea = (bx1 - bx0) * (by1 - by0)
    parea = (px1 - px0) * (py1 - py0)
    ov = inter / (barea + parea - inter)

    jj = lax.broadcasted_iota(_i32, (NOBJ, 1), 0)
    nn = lax.broadcasted_iota(_i32, (1, N), 1)

    ofp = jnp.max(ov, axis=0, keepdims=True)
    obj = jnp.min(jnp.where(ov == ofp, jj, NOBJ), axis=0, keepdims=True)

    oeo = jnp.max(ov, axis=1, keepdims=True)
    pfe = jnp.min(jnp.where(ov == oeo, nn, N), axis=1, keepdims=True)
    maskj = oeo > 0.0

    cond = (nn == pfe) & maskj
    bestj = jnp.max(jnp.where(cond, jj, -1), axis=0, keepdims=True)
    forced = bestj >= 0
    ofp = jnp.where(forced, 1.0, ofp)
    obj = jnp.where(forced, bestj, obj)

    selF = (jj == obj).astype(_f32)
    bxT = jnp.transpose(bx)
    labT = jnp.transpose(labels_ref[0].astype(_f32))
    tbl = jnp.concatenate([bxT, labT], axis=0)
    gath = jax.lax.dot_general(tbl, selF, (((1,), (0,)), ((), ())),
                               preferred_element_type=_f32)
    gx0, gy0 = gath[0:1], gath[1:2]
    gx1, gy1 = gath[2:3], gath[3:4]
    labf = gath[4:5]

    labf = jnp.where(ofp < THRESHOLD, -1.0, labf)
    labf = jnp.where(ofp < THRESHOLD - 0.1, 0.0, labf)

    ex = (0.5 * (gx0 + gx1) - pcx) / (pw / 10.0)
    ey = (0.5 * (gy0 + gy1) - pcy) / (ph / 10.0)
    ew = jnp.log(jnp.maximum((gx1 - gx0) / pw, 1e-12)) * 5.0
    eh = jnp.log(jnp.maximum((gy1 - gy0) / ph, 1e-12)) * 5.0
    enc = jnp.concatenate([ex, ey, ew, eh], axis=0)

    posf = (labf > 0.5).astype(_f32)
    d = plocs_ref[0] - enc
    ad = jnp.abs(d)
    sl1 = jnp.where(ad < 1.0, 0.5 * d * d, ad - 0.5)
    loc_part = jnp.sum(sl1 * posf)
    npos_part = jnp.sum(posf)

    tcls_ref[0] = labf

    @pl.when(b == 0)
    def _():
        loc_ref[0, 0] = 0.0
        npos_ref[0, 0] = 0.0

    loc_ref[0, 0] += loc_part
    npos_ref[0, 0] += npos_part


def _conf_half(s_ref, tsc, off):
    s = s_ref[0]
    tcf = tsc[pl.ds(off, TP), :].astype(_f32)

    e = jnp.exp(-s)
    p = 1.0 / (1.0 + e)
    omp = 1.0 - p
    term2 = p * p * jnp.log(jnp.maximum(omp, 1e-12))
    s_row = jnp.sum(term2, axis=1, keepdims=True)

    crf = (lax.broadcasted_iota(_i32, (1, C), 1) + 1).astype(_f32)
    pos = tcf == crf
    xk_col = jnp.sum(jnp.where(pos, s, 0.0), axis=1, keepdims=True)

    pack = jnp.concatenate([s_row, xk_col, tcf], axis=1)
    pkT = jnp.transpose(pack)
    srow_l, xk_l, t_l = pkT[0:1], pkT[1:2], pkT[2:3]

    sum_valid = jnp.sum(jnp.where(t_l >= 0.0, srow_l, 0.0))

    e2 = jnp.exp(-xk_l)
    p2 = 1.0 / (1.0 + e2)
    om2 = 1.0 - p2
    t1k = (om2 * om2) * jnp.log(jnp.maximum(p2, 1e-12))
    t2k = (p2 * p2) * jnp.log(jnp.maximum(om2, 1e-12))
    corr = jnp.sum(jnp.where(t_l > 0.5,
                             (1.0 - ALPHA_F) * t2k - ALPHA_F * t1k, 0.0))
    return -(1.0 - ALPHA_F) * sum_valid + corr


def _conf_body(s1_ref, s2_ref, t1_ref, t2_ref, out_ref, tsc1, tsc2):
    b = pl.program_id(0)
    t = pl.program_id(1)

    @pl.when(t == 0)
    def _():
        tsc1[...] = jnp.transpose(t1_ref[0].astype(_bf16))
        tsc2[...] = jnp.transpose(t2_ref[0].astype(_bf16))

    off = pl.multiple_of(t * TP, 8)
    part = _conf_half(s1_ref, tsc1, off) + _conf_half(s2_ref, tsc2, off)

    @pl.when(jnp.logical_and(b == 0, t == 0))
    def _():
        out_ref[0, 0] = 0.0

    out_ref[0, 0] += part


def kernel(predicted_locs, predicted_scores, boxes, labels, priors_cxcy):
    labels3 = labels.astype(_i32).reshape(B, NOBJ, 1)
    priors_t = priors_cxcy.T
    plocs_t = jnp.transpose(predicted_locs, (0, 2, 1))

    tcls, loc_sum, npos = pl.pallas_call(
        _match_body,
        grid=(B,),
        in_specs=[
            pl.BlockSpec((1, NOBJ, 4), lambda b: (b, 0, 0)),
            pl.BlockSpec((1, NOBJ, 1), lambda b: (b, 0, 0)),
            pl.BlockSpec((4, N), lambda b: (0, 0)),
            pl.BlockSpec((1, 4, N), lambda b: (b, 0, 0)),
        ],
        out_specs=[
            pl.BlockSpec((1, 1, N), lambda b: (b, 0, 0)),
            pl.BlockSpec((1, 1), lambda b: (0, 0), memory_space=pltpu.SMEM),
            pl.BlockSpec((1, 1), lambda b: (0, 0), memory_space=pltpu.SMEM),
        ],
        out_shape=[
            jax.ShapeDtypeStruct((B, 1, N), _f32),
            jax.ShapeDtypeStruct((1, 1), _f32),
            jax.ShapeDtypeStruct((1, 1), _f32),
        ],
    )(boxes, labels3, priors_t, plocs_t)

    conf = pl.pallas_call(
        _conf_body,
        grid=(B // 2, NT),
        in_specs=[
            pl.BlockSpec((1, TP, C), lambda b, t: (b, t, 0)),
            pl.BlockSpec((1, TP, C), lambda b, t: (b + B // 2, t, 0)),
            pl.BlockSpec((1, 1, N), lambda b, t: (b, 0, 0)),
            pl.BlockSpec((1, 1, N), lambda b, t: (b + B // 2, 0, 0)),
        ],
        out_specs=pl.BlockSpec((1, 1), lambda b, t: (0, 0),
                               memory_space=pltpu.SMEM),
        out_shape=jax.ShapeDtypeStruct((1, 1), _f32),
        scratch_shapes=[pltpu.VMEM((N, 1), _bf16), pltpu.VMEM((N, 1), _bf16)],
    )(predicted_scores, predicted_scores, tcls, tcls)

    np_ = npos[0, 0]
    return conf[0, 0] / np_ + loc_sum[0, 0] / (np_ * 4.0)


# speedup vs baseline: 3.6359x; 1.1251x over previous
import jax
import jax.numpy as jnp
from jax import lax
from jax.experimental import pallas as pl
from jax.experimental.pallas import tpu as pltpu

B = 8
N = 22536
NOBJ = 16
C = 80
THRESHOLD = 0.5
ALPHA_F = 0.25
TP = 7512
NT = N // TP

_f32 = jnp.float32
_i32 = jnp.int32
_bf16 = jnp.bfloat16


def _match_compute(bx, labs, pr, plocs):
    pcx, pcy = pr[0:1], pr[1:2]
    pw, ph = pr[2:3], pr[3:4]
    px0 = pcx - pw * 0.5
    py0 = pcy - ph * 0.5
    px1 = pcx + pw * 0.5
    py1 = pcy + ph * 0.5

    bx0, by0 = bx[:, 0:1], bx[:, 1:2]
    bx1, by1 = bx[:, 2:3], bx[:, 3:4]

    iw = jnp.maximum(jnp.minimum(bx1, px1) - jnp.maximum(bx0, px0), 0.0)
    ih = jnp.maximum(jnp.minimum(by1, py1) - jnp.maximum(by0, py0), 0.0)
    inter = iw * ih
    barea = (bx1 - bx0) * (by1 - by0)
    parea = (px1 - px0) * (py1 - py0)
    ov = inter / (barea + parea - inter)

    jj = lax.broadcasted_iota(_i32, (NOBJ, 1), 0)
    nn = lax.broadcasted_iota(_i32, (1, N), 1)

    ofp = jnp.max(ov, axis=0, keepdims=True)
    obj = jnp.min(jnp.where(ov == ofp, jj, NOBJ), axis=0, keepdims=True)

    oeo = jnp.max(ov, axis=1, keepdims=True)
    pfe = jnp.min(jnp.where(ov == oeo, nn, N), axis=1, keepdims=True)
    maskj = oeo > 0.0

    cond = (nn == pfe) & maskj
    bestj = jnp.max(jnp.where(cond, jj, -1), axis=0, keepdims=True)
    forced = bestj >= 0
    ofp = jnp.where(forced, 1.0, ofp)
    obj = jnp.where(forced, bestj, obj)

    selF = (jj == obj).astype(_f32)
    bxT = jnp.transpose(bx)
    labT = jnp.transpose(labs)
    tbl = jnp.concatenate([bxT, labT], axis=0)
    gath = jax.lax.dot_general(tbl, selF, (((1,), (0,)), ((), ())),
                               preferred_element_type=_f32)
    gx0, gy0 = gath[0:1], gath[1:2]
    gx1, gy1 = gath[2:3], gath[3:4]
    labf = gath[4:5]

    labf = jnp.where(ofp < THRESHOLD, -1.0, labf)
    labf = jnp.where(ofp < THRESHOLD - 0.1, 0.0, labf)

    ex = (0.5 * (gx0 + gx1) - pcx) / (pw / 10.0)
    ey = (0.5 * (gy0 + gy1) - pcy) / (ph / 10.0)
    ew = jnp.log(jnp.maximum((gx1 - gx0) / pw, 1e-12)) * 5.0
    eh = jnp.log(jnp.maximum((gy1 - gy0) / ph, 1e-12)) * 5.0
    enc = jnp.concatenate([ex, ey, ew, eh], axis=0)

    posf = (labf > 0.5).astype(_f32)
    d = plocs - enc
    ad = jnp.abs(d)
    sl1 = jnp.where(ad < 1.0, 0.5 * d * d, ad - 0.5)
    loc_part = jnp.sum(sl1 * posf)
    npos_part = jnp.sum(posf)
    return labf, loc_part, npos_part


def _conf_half(s_ref, tsc, off):
    s = s_ref[0]
    tcf = tsc[pl.ds(off, TP), :].astype(_f32)

    crf = (lax.broadcasted_iota(_i32, (1, C), 1) + 1).astype(_f32)
    pos = tcf == crf
    neg = (~pos) & (tcf >= 0.0)

    e = jnp.exp(-s)
    u = 1.0 + e
    el = jnp.log(u)
    q = jnp.exp(-2.0 * el)
    lo = -s - el
    term2 = q * lo
    term1 = (e * e * q) * (-el)
    loss = (jnp.where(pos, -ALPHA_F * term1, 0.0)
            + jnp.where(neg, (ALPHA_F - 1.0) * term2, 0.0))
    return jnp.sum(loss)


def _fused_body(s1_ref, s2_ref, bx1_ref, bx2_ref, lb1_ref, lb2_ref,
                pri_ref, pl1_ref, pl2_ref,
                conf_ref, loc_ref, npos_ref, tsc1, tsc2):
    b = pl.program_id(0)
    t = pl.program_id(1)

    @pl.when(jnp.logical_and(b == 0, t == 0))
    def _():
        conf_ref[0, 0] = 0.0
        loc_ref[0, 0] = 0.0
        npos_ref[0, 0] = 0.0

    @pl.when(t == 0)
    def _():
        pr = pri_ref[...]
        lab1, lp1, np1 = _match_compute(
            bx1_ref[0], lb1_ref[0].astype(_f32), pr, pl1_ref[0])
        lab2, lp2, np2 = _match_compute(
            bx2_ref[0], lb2_ref[0].astype(_f32), pr, pl2_ref[0])
        tsc1[...] = jnp.transpose(lab1.astype(_bf16))
        tsc2[...] = jnp.transpose(lab2.astype(_bf16))
        loc_ref[0, 0] += lp1 + lp2
        npos_ref[0, 0] += np1 + np2

    off = pl.multiple_of(t * TP, 8)
    conf_ref[0, 0] += _conf_half(s1_ref, tsc1, off) + _conf_half(s2_ref, tsc2, off)


def kernel(predicted_locs, predicted_scores, boxes, labels, priors_cxcy):
    labels3 = labels.astype(_i32).reshape(B, NOBJ, 1)
    priors_t = priors_cxcy.T
    plocs_t = jnp.transpose(predicted_locs, (0, 2, 1))
    B2 = B // 2

    conf, loc_sum, npos = pl.pallas_call(
        _fused_body,
        grid=(B2, NT),
        in_specs=[
            pl.BlockSpec((1, TP, C), lambda b, t: (b, t, 0)),
            pl.BlockSpec((1, TP, C), lambda b, t: (b + B2, t, 0)),
            pl.BlockSpec((1, NOBJ, 4), lambda b, t: (b, 0, 0)),
            pl.BlockSpec((1, NOBJ, 4), lambda b, t: (b + B2, 0, 0)),
            pl.BlockSpec((1, NOBJ, 1), lambda b, t: (b, 0, 0)),
            pl.BlockSpec((1, NOBJ, 1), lambda b, t: (b + B2, 0, 0)),
            pl.BlockSpec((4, N), lambda b, t: (0, 0)),
            pl.BlockSpec((1, 4, N), lambda b, t: (b, 0, 0)),
            pl.BlockSpec((1, 4, N), lambda b, t: (b + B2, 0, 0)),
        ],
        out_specs=[
            pl.BlockSpec((1, 1), lambda b, t: (0, 0), memory_space=pltpu.SMEM),
            pl.BlockSpec((1, 1), lambda b, t: (0, 0), memory_space=pltpu.SMEM),
            pl.BlockSpec((1, 1), lambda b, t: (0, 0), memory_space=pltpu.SMEM),
        ],
        out_shape=[
            jax.ShapeDtypeStruct((1, 1), _f32),
            jax.ShapeDtypeStruct((1, 1), _f32),
            jax.ShapeDtypeStruct((1, 1), _f32),
        ],
        scratch_shapes=[pltpu.VMEM((N, 1), _bf16), pltpu.VMEM((N, 1), _bf16)],
    )(predicted_scores, predicted_scores, boxes, boxes, labels3, labels3,
      priors_t, plocs_t, plocs_t)

    np_ = npos[0, 0]
    return conf[0, 0] / np_ + loc_sum[0, 0] / (np_ * 4.0)
